# Initial kernel scaffold; baseline (speedup 1.0000x reference)
import jax
import jax.numpy as jnp
from jax.experimental import pallas as pl


def kernel(center_table, context_table, center_word_indices, context_word_indices, negative_word_indices):
    raise NotImplementedError("write your pallas kernel here")


# speedup vs baseline: 2.3541x; 2.3541x over previous
import functools

import jax
import jax.numpy as jnp
from jax import lax
from jax.experimental import pallas as pl
from jax.experimental.pallas import tpu as pltpu
from jax.experimental.pallas import tpu_sc as plsc

B = 16384
D = 64
K = 20
NC = 2
NS = 16
L = 16
NW = NC * NS
PER_W = B // NW
C = 32
NCHUNK = PER_W // C
NEG_IDX_GROUPS = (C * K) // 128


def _sc_body(center_tbl, context_tbl, cidx_hbm, xidx_hbm, nidx_hbm,
             pos_hbm, neg_hbm,
             cidx_v, xidx_v, nidx_v, crows, xrows, nrows, posbuf, negbuf,
             sem):
  wid = lax.axis_index("c") * NS + lax.axis_index("s")
  base = wid * PER_W

  pltpu.sync_copy(cidx_hbm.at[wid], cidx_v)
  pltpu.sync_copy(xidx_hbm.at[wid], xidx_v)
  pltpu.sync_copy(nidx_hbm.at[wid], nidx_v)

  def chunk_body(i, carry):
    cps = [
        pltpu.async_copy(center_tbl.at[cidx_v.at[i]], crows, sem),
        pltpu.async_copy(context_tbl.at[xidx_v.at[i]], xrows, sem),
    ]
    for j in range(NEG_IDX_GROUPS):
      cps.append(pltpu.async_copy(context_tbl.at[nidx_v.at[i, j]],
                                  nrows.at[pl.ds(j * 128, 128)], sem))
    for cp in cps:
      cp.wait()

    for g in range(C // L):
      eids = g * L + lax.iota(jnp.int32, L)

      def dbody(d, accs):
        dv = jnp.full((L,), d, jnp.int32)
        c = plsc.load_gather(crows, [eids, dv])
        x = plsc.load_gather(xrows, [eids, dv])
        out = [accs[0] + c * x]
        for k in range(K):
          n = plsc.load_gather(nrows, [eids * K + k, dv])
          out.append(accs[k + 1] + c * n)
        return tuple(out)

      accs = lax.fori_loop(
          0, D, dbody,
          tuple(jnp.zeros((L,), jnp.float32) for _ in range(K + 1)))
      posbuf[pl.ds(g * L, L)] = accs[0]
      for k in range(K):
        plsc.store_scatter(negbuf, [eids, jnp.full((L,), k, jnp.int32)],
                           accs[k + 1])

    pltpu.sync_copy(posbuf, pos_hbm.at[pl.ds(base + i * C, C)])
    pltpu.sync_copy(negbuf, neg_hbm.at[pl.ds(base + i * C, C)])
    return carry

  lax.fori_loop(0, NCHUNK, chunk_body, 0)


_sc_scores = functools.partial(
    pl.kernel,
    out_type=(
        jax.ShapeDtypeStruct((B,), jnp.float32),
        jax.ShapeDtypeStruct((B, K), jnp.float32),
    ),
    mesh=plsc.VectorSubcoreMesh(core_axis_name="c", subcore_axis_name="s"),
    scratch_types=(
        pltpu.VMEM((NCHUNK, C), jnp.int32),
        pltpu.VMEM((NCHUNK, C), jnp.int32),
        pltpu.VMEM((NCHUNK, NEG_IDX_GROUPS, 128), jnp.int32),
        pltpu.VMEM((C, D), jnp.float32),
        pltpu.VMEM((C, D), jnp.float32),
        pltpu.VMEM((C * K, D), jnp.float32),
        pltpu.VMEM((C,), jnp.float32),
        pltpu.VMEM((C, K), jnp.float32),
        pltpu.SemaphoreType.DMA,
    ),
    compiler_params=pltpu.CompilerParams(needs_layout_passes=False,
                                         use_tc_tiling_on_sc=False),
)(_sc_body)


def _log_sigmoid(x):
  return jnp.minimum(x, 0.0) - jnp.log1p(jnp.exp(-jnp.abs(x)))


def _loss_body(pos_ref, neg_ref, out_ref):
  t1 = jnp.sum(_log_sigmoid(pos_ref[...]))
  t2 = jnp.sum(_log_sigmoid(-neg_ref[...]))
  loss = -(t1 + t2) / jnp.float32(2 * B)
  out_ref[...] = jnp.full((1, 1), loss, jnp.float32)


_loss_call = pl.pallas_call(
    _loss_body,
    out_shape=jax.ShapeDtypeStruct((1, 1), jnp.float32),
)


@jax.jit
def kernel(center_table, context_table, center_word_indices,
           context_word_indices, negative_word_indices):
  cidx = center_word_indices.astype(jnp.int32).reshape(NW, NCHUNK, C)
  xidx = context_word_indices.astype(jnp.int32).reshape(NW, NCHUNK, C)
  nidx = negative_word_indices.astype(jnp.int32).reshape(
      NW, NCHUNK, NEG_IDX_GROUPS, 128)
  pos, neg = _sc_scores(center_table, context_table, cidx, xidx, nidx)
  loss = _loss_call(pos.reshape(B // 128, 128),
                    neg.reshape((B * K) // 128, 128))
  return loss[0, 0]
